# baseline (speedup 1.0000x reference)
import functools

import jax
import jax.numpy as jnp
from jax import lax
from jax.experimental import pallas as pl
from jax.experimental.pallas import tpu as pltpu
from jax.experimental.pallas import tpu_sc as plsc

T, K, E, C, D, F = 2048, 2, 8, 1024, 1024, 4096
TB = 64
NB = T // TB
NW = 32
TOK_W = T // NW
SENT = E * C
XROWS = E * C + 8
BF = 1024
NF = F // BF


def _routing_body(idx_ref, aff_ref, slot0_ref, slot1_ref, a0_ref, a1_ref,
                  cnt_ref):
    idx = idx_ref[...]
    aff = aff_ref[...]
    e_iota = lax.broadcasted_iota(jnp.int32, (1, E), 1)
    rows_i = lax.broadcasted_iota(jnp.int32, (TB, TB), 0)
    cols_i = lax.broadcasted_iota(jnp.int32, (TB, TB), 1)
    lincl = (cols_i <= rows_i).astype(jnp.float32)
    ident = (cols_i == rows_i).astype(jnp.float32)

    del ident
    carry = jnp.zeros((1, E), jnp.float32)
    for b in range(NB):
        sl = slice(b * TB, (b + 1) * TB)
        i0 = idx[sl, 0:1]
        i1 = idx[sl, 1:2]
        oh0 = (i0 == e_iota).astype(jnp.float32)
        oh1 = (i1 == e_iota).astype(jnp.float32)
        c0 = jnp.dot(lincl, oh0, preferred_element_type=jnp.float32)
        c1 = jnp.dot(lincl, oh1, preferred_element_type=jnp.float32)
        base = carry + c0 + c1 - oh0 - oh1
        pos0 = jnp.sum(base * oh0, axis=1, keepdims=True)
        pos1 = jnp.sum((base + oh0) * oh1, axis=1, keepdims=True)
        carry = carry + c0[TB - 1:TB, :] + c1[TB - 1:TB, :]
        v0 = pos0 < C
        v1 = pos1 < C
        s0 = jnp.where(v0, i0.astype(jnp.float32) * C + pos0, float(SENT))
        s1 = jnp.where(v1, i1.astype(jnp.float32) * C + pos1, float(SENT))
        ab = aff[sl, :]
        t0 = jnp.sum(ab * oh0, axis=1, keepdims=True)
        t1 = jnp.sum(ab * oh1, axis=1, keepdims=True)
        ssum = t0 + t1 + 1e-9
        a0 = jnp.where(v0, t0 / ssum, 0.0)
        a1 = jnp.where(v1, t1 / ssum, 0.0)
        slot0_ref[sl, :] = jnp.broadcast_to(s0.astype(jnp.int32), (TB, 16))
        slot1_ref[sl, :] = jnp.broadcast_to(s1.astype(jnp.int32), (TB, 16))
        a0_ref[sl, :] = jnp.broadcast_to(a0, (TB, 16))
        a1_ref[sl, :] = jnp.broadcast_to(a1, (TB, 16))
    cnt_ref[...] = carry.astype(jnp.int32)


def _routing(expert_indices, expert_affinities):
    return pl.pallas_call(
        _routing_body,
        out_shape=(
            jax.ShapeDtypeStruct((T, 16), jnp.int32),
            jax.ShapeDtypeStruct((T, 16), jnp.int32),
            jax.ShapeDtypeStruct((T, 16), jnp.float32),
            jax.ShapeDtypeStruct((T, 16), jnp.float32),
            jax.ShapeDtypeStruct((1, E), jnp.int32),
        ),
    )(expert_indices, expert_affinities)


def _dispatch_body(hid_hbm, slot0_hbm, slot1_hbm, xe_hbm, s0_v, s1_v, rows_v,
                   sem0, sem1):
    wid = lax.axis_index("s") * 2 + lax.axis_index("c")
    tok0 = wid * TOK_W
    pltpu.sync_copy(slot0_hbm.at[pl.ds(tok0, TOK_W)], s0_v)
    pltpu.sync_copy(slot1_hbm.at[pl.ds(tok0, TOK_W)], s1_v)
    for cidx in range(TOK_W // 16):
        tb = tok0 + cidx * 16
        pltpu.sync_copy(hid_hbm.at[pl.ds(tb, 16)], rows_v)
        i0 = s0_v[pl.ds(cidx * 16, 16)]
        i1 = s1_v[pl.ds(cidx * 16, 16)]
        cp0 = pltpu.async_copy(rows_v, xe_hbm.at[i0], sem0)
        cp1 = pltpu.async_copy(rows_v, xe_hbm.at[i1], sem1)
        cp0.wait()
        cp1.wait()


def _dispatch(hidden_states, slot0, slot1):
    mesh = plsc.VectorSubcoreMesh(core_axis_name="c", subcore_axis_name="s",
                                  num_cores=2, num_subcores=16)
    f = functools.partial(
        pl.kernel,
        out_type=jax.ShapeDtypeStruct((XROWS, D), jnp.float32),
        mesh=mesh,
        scratch_types=[
            pltpu.VMEM((TOK_W,), jnp.int32),
            pltpu.VMEM((TOK_W,), jnp.int32),
            pltpu.VMEM((16, D), jnp.float32),
            pltpu.SemaphoreType.DMA,
            pltpu.SemaphoreType.DMA,
        ],
    )(_dispatch_body)
    return f(hidden_states, slot0, slot1)


def _mlp_body(x_ref, wg_ref, wu_ref, wd_ref, o_ref):
    f = pl.program_id(1)
    x = x_ref[...].astype(jnp.bfloat16)
    g = jnp.dot(x, wg_ref[0], preferred_element_type=jnp.float32)
    u = jnp.dot(x, wu_ref[0], preferred_element_type=jnp.float32)
    a = (g * lax.logistic(g) * u).astype(jnp.bfloat16)
    o = jnp.dot(a, wd_ref[0], preferred_element_type=jnp.float32)

    @pl.when(f == 0)
    def _init():
        o_ref[...] = o

    @pl.when(f != 0)
    def _acc():
        o_ref[...] = o_ref[...] + o


def _mlp(xe, gate_up_proj, down_proj):
    return pl.pallas_call(
        _mlp_body,
        grid=(E, NF),
        in_specs=[
            pl.BlockSpec((C, D), lambda e, f: (e, 0)),
            pl.BlockSpec((1, D, BF), lambda e, f: (e, 0, f)),
            pl.BlockSpec((1, D, BF), lambda e, f: (e, 0, NF + f)),
            pl.BlockSpec((1, BF, D), lambda e, f: (e, f, 0)),
        ],
        out_specs=pl.BlockSpec((C, D), lambda e, f: (e, 0)),
        out_shape=jax.ShapeDtypeStruct((E * C, D), jnp.float32),
        compiler_params=pltpu.CompilerParams(
            dimension_semantics=("parallel", "arbitrary")),
    )(xe, gate_up_proj, gate_up_proj, down_proj)


def _combine_body(oute_hbm, slot0_hbm, slot1_hbm, a0_hbm, a1_hbm, y_hbm,
                  s0_v, s1_v, a0_v, a1_v, r0_v, r1_v, y_v, sem0, sem1):
    wid = lax.axis_index("s") * 2 + lax.axis_index("c")
    tok0 = wid * TOK_W
    pltpu.sync_copy(slot0_hbm.at[pl.ds(tok0, TOK_W)], s0_v)
    pltpu.sync_copy(slot1_hbm.at[pl.ds(tok0, TOK_W)], s1_v)
    pltpu.sync_copy(a0_hbm.at[pl.ds(tok0, TOK_W)], a0_v)
    pltpu.sync_copy(a1_hbm.at[pl.ds(tok0, TOK_W)], a1_v)
    for cidx in range(TOK_W // 16):
        i0 = jnp.minimum(s0_v[pl.ds(cidx * 16, 16)], E * C - 1)
        i1 = jnp.minimum(s1_v[pl.ds(cidx * 16, 16)], E * C - 1)
        cp0 = pltpu.async_copy(oute_hbm.at[i0], r0_v, sem0)
        cp1 = pltpu.async_copy(oute_hbm.at[i1], r1_v, sem1)
        cp0.wait()
        cp1.wait()
        for i in range(16):
            p = cidx * 16 + i
            a0 = a0_v[p, :]
            a1 = a1_v[p, :]
            zero = jnp.zeros((16,), jnp.float32)

            def body(cc, _):
                r0 = r0_v[i, pl.ds(cc * 16, 16)]
                r1 = r1_v[i, pl.ds(cc * 16, 16)]
                y_v[i, pl.ds(cc * 16, 16)] = (
                    jnp.where(a0 > 0, a0 * r0, zero)
                    + jnp.where(a1 > 0, a1 * r1, zero))
                return 0

            lax.fori_loop(0, D // 16, body, 0)
        pltpu.sync_copy(y_v, y_hbm.at[pl.ds(tok0 + cidx * 16, 16)])


def _combine(out_e, slot0, slot1, a0, a1):
    mesh = plsc.VectorSubcoreMesh(core_axis_name="c", subcore_axis_name="s",
                                  num_cores=2, num_subcores=16)
    f = functools.partial(
        pl.kernel,
        out_type=jax.ShapeDtypeStruct((T, D), jnp.float32),
        mesh=mesh,
        scratch_types=[
            pltpu.VMEM((TOK_W,), jnp.int32),
            pltpu.VMEM((TOK_W,), jnp.int32),
            pltpu.VMEM((TOK_W, 16), jnp.float32),
            pltpu.VMEM((TOK_W, 16), jnp.float32),
            pltpu.VMEM((16, D), jnp.float32),
            pltpu.VMEM((16, D), jnp.float32),
            pltpu.VMEM((16, D), jnp.float32),
            pltpu.SemaphoreType.DMA,
            pltpu.SemaphoreType.DMA,
        ],
    )(_combine_body)
    return f(out_e, slot0, slot1, a0, a1)


def kernel(hidden_states, expert_affinities, expert_indices, seq_len,
           gate_up_proj, down_proj):
    del seq_len
    ps0, ps1, a0, a1, _cnt = _routing(expert_indices, expert_affinities)
    slot0 = ps0[:, 0]
    slot1 = ps1[:, 0]
    xe = _dispatch(hidden_states, slot0, slot1)
    out_e = _mlp(xe, gate_up_proj.astype(jnp.bfloat16),
                 down_proj.astype(jnp.bfloat16))
    return _combine(out_e, slot0, slot1, a0, a1)


def _routing_xla(expert_indices, expert_affinities):
    flat_idx = expert_indices.reshape(-1)
    onehot = jax.nn.one_hot(flat_idx, E, dtype=jnp.int32)
    pos = jnp.sum((jnp.cumsum(onehot, axis=0) - 1) * onehot, axis=1)
    valid = pos < C
    slot = jnp.where(valid, flat_idx * C + pos, SENT).astype(jnp.int32)
    topk = jnp.take_along_axis(expert_affinities, expert_indices, axis=1)
    topk = topk / (jnp.sum(topk, axis=1, keepdims=True) + 1e-9)
    aff = (topk * valid.reshape(T, K)).reshape(-1)
    s2 = slot.reshape(T, K)
    a2 = aff.reshape(T, K)
    a0 = jnp.broadcast_to(a2[:, 0:1], (T, 16))
    a1 = jnp.broadcast_to(a2[:, 1:2], (T, 16))
    return s2[:, 0], s2[:, 1], a0, a1


def _dispatch_xla(hidden, s0, s1):
    xe = jnp.zeros((XROWS, D), jnp.float32)
    xe = xe.at[s0].set(hidden)
    xe = xe.at[s1].set(hidden)
    return xe


def _combine_xla(out_e, s0, s1, a0b, a1b):
    a0 = a0b[:, 0]
    a1 = a1b[:, 0]
    i0 = jnp.minimum(s0, E * C - 1)
    i1 = jnp.minimum(s1, E * C - 1)
    r0 = out_e[i0] * jnp.where(a0 > 0, a0, 0.0)[:, None]
    r1 = out_e[i1] * jnp.where(a1 > 0, a1, 0.0)[:, None]
    return r0 + r1


# speedup vs baseline: 1.7964x; 1.7964x over previous
import functools

import jax
import jax.numpy as jnp
from jax import lax
from jax.experimental import pallas as pl
from jax.experimental.pallas import tpu as pltpu
from jax.experimental.pallas import tpu_sc as plsc

T, K, E, C, D, F = 2048, 2, 8, 1024, 1024, 4096
TB = 64
NB = T // TB
NW = 32
TOK_W = T // NW
SENT = E * C
XROWS = E * C + 8
BF = 1024
NF = F // BF


def _routing_body(idx_ref, aff_ref, slot0_ref, slot1_ref, a0_ref, a1_ref,
                  cnt_ref):
    idx = idx_ref[...]
    aff = aff_ref[...]
    e_iota = lax.broadcasted_iota(jnp.int32, (1, E), 1)
    rows_i = lax.broadcasted_iota(jnp.int32, (TB, TB), 0)
    cols_i = lax.broadcasted_iota(jnp.int32, (TB, TB), 1)
    lincl = (cols_i <= rows_i).astype(jnp.float32)
    ident = (cols_i == rows_i).astype(jnp.float32)

    del ident
    carry = jnp.zeros((1, E), jnp.float32)
    for b in range(NB):
        sl = slice(b * TB, (b + 1) * TB)
        i0 = idx[sl, 0:1]
        i1 = idx[sl, 1:2]
        oh0 = (i0 == e_iota).astype(jnp.float32)
        oh1 = (i1 == e_iota).astype(jnp.float32)
        c0 = jnp.dot(lincl, oh0, preferred_element_type=jnp.float32)
        c1 = jnp.dot(lincl, oh1, preferred_element_type=jnp.float32)
        base = carry + c0 + c1 - oh0 - oh1
        pos0 = jnp.sum(base * oh0, axis=1, keepdims=True)
        pos1 = jnp.sum((base + oh0) * oh1, axis=1, keepdims=True)
        carry = carry + c0[TB - 1:TB, :] + c1[TB - 1:TB, :]
        v0 = pos0 < C
        v1 = pos1 < C
        s0 = jnp.where(v0, i0.astype(jnp.float32) * C + pos0, float(SENT))
        s1 = jnp.where(v1, i1.astype(jnp.float32) * C + pos1, float(SENT))
        ab = aff[sl, :]
        t0 = jnp.sum(ab * oh0, axis=1, keepdims=True)
        t1 = jnp.sum(ab * oh1, axis=1, keepdims=True)
        ssum = t0 + t1 + 1e-9
        a0 = jnp.where(v0, t0 / ssum, 0.0)
        a1 = jnp.where(v1, t1 / ssum, 0.0)
        slot0_ref[sl, :] = jnp.broadcast_to(s0.astype(jnp.int32), (TB, 16))
        slot1_ref[sl, :] = jnp.broadcast_to(s1.astype(jnp.int32), (TB, 16))
        a0_ref[sl, :] = jnp.broadcast_to(a0, (TB, 16))
        a1_ref[sl, :] = jnp.broadcast_to(a1, (TB, 16))
    cnt_ref[...] = carry.astype(jnp.int32)


def _routing(expert_indices, expert_affinities):
    return pl.pallas_call(
        _routing_body,
        out_shape=(
            jax.ShapeDtypeStruct((T, 16), jnp.int32),
            jax.ShapeDtypeStruct((T, 16), jnp.int32),
            jax.ShapeDtypeStruct((T, 16), jnp.float32),
            jax.ShapeDtypeStruct((T, 16), jnp.float32),
            jax.ShapeDtypeStruct((1, E), jnp.int32),
        ),
    )(expert_indices, expert_affinities)


def _dispatch_body(hid_hbm, slot0_hbm, slot1_hbm, xe_hbm, s0_v, s1_v, rows_v,
                   sem0, sem1):
    wid = lax.axis_index("s") * 2 + lax.axis_index("c")
    tok0 = wid * TOK_W
    pltpu.sync_copy(slot0_hbm.at[pl.ds(tok0, TOK_W)], s0_v)
    pltpu.sync_copy(slot1_hbm.at[pl.ds(tok0, TOK_W)], s1_v)
    for cidx in range(TOK_W // 16):
        tb = tok0 + cidx * 16
        pltpu.sync_copy(hid_hbm.at[pl.ds(tb, 16)], rows_v)
        i0 = s0_v[pl.ds(cidx * 16, 16)]
        i1 = s1_v[pl.ds(cidx * 16, 16)]
        cp0 = pltpu.async_copy(rows_v, xe_hbm.at[i0], sem0)
        cp1 = pltpu.async_copy(rows_v, xe_hbm.at[i1], sem1)
        cp0.wait()
        cp1.wait()


def _dispatch(hidden_states, slot0, slot1):
    mesh = plsc.VectorSubcoreMesh(core_axis_name="c", subcore_axis_name="s",
                                  num_cores=2, num_subcores=16)
    f = functools.partial(
        pl.kernel,
        out_type=jax.ShapeDtypeStruct((XROWS, D), jnp.float32),
        mesh=mesh,
        scratch_types=[
            pltpu.VMEM((TOK_W,), jnp.int32),
            pltpu.VMEM((TOK_W,), jnp.int32),
            pltpu.VMEM((16, D), jnp.float32),
            pltpu.SemaphoreType.DMA,
            pltpu.SemaphoreType.DMA,
        ],
    )(_dispatch_body)
    return f(hidden_states, slot0, slot1)


BC = 128
RC = C // BC


def _mlp_body(cnt_ref, x_ref, wg_ref, wu_ref, wd_ref, o_ref):
    e = pl.program_id(0)
    f = pl.program_id(1)
    cnt = cnt_ref[e]
    wg = wg_ref[0].astype(jnp.bfloat16)
    wu = wu_ref[0].astype(jnp.bfloat16)
    wd = wd_ref[0].astype(jnp.bfloat16)
    for rc in range(RC):
        @pl.when(cnt > rc * BC)
        def _chunk(rc=rc):
            sl = slice(rc * BC, (rc + 1) * BC)
            x = x_ref[sl, :].astype(jnp.bfloat16)
            g = jnp.dot(x, wg, preferred_element_type=jnp.float32)
            u = jnp.dot(x, wu, preferred_element_type=jnp.float32)
            a = (g * lax.logistic(g) * u).astype(jnp.bfloat16)
            o = jnp.dot(a, wd, preferred_element_type=jnp.float32)

            @pl.when(f == 0)
            def _init():
                o_ref[sl, :] = o

            @pl.when(f != 0)
            def _acc():
                o_ref[sl, :] = o_ref[sl, :] + o


def _mlp(cnt, xe, gate_up_proj, down_proj):
    grid_spec = pltpu.PrefetchScalarGridSpec(
        num_scalar_prefetch=1,
        grid=(E, NF),
        in_specs=[
            pl.BlockSpec((C, D), lambda e, f, cnt: (e, 0)),
            pl.BlockSpec((1, D, BF), lambda e, f, cnt: (e, 0, f)),
            pl.BlockSpec((1, D, BF), lambda e, f, cnt: (e, 0, NF + f)),
            pl.BlockSpec((1, BF, D), lambda e, f, cnt: (e, f, 0)),
        ],
        out_specs=pl.BlockSpec((C, D), lambda e, f, cnt: (e, 0)),
    )
    return pl.pallas_call(
        _mlp_body,
        grid_spec=grid_spec,
        out_shape=jax.ShapeDtypeStruct((E * C, D), jnp.float32),
        compiler_params=pltpu.CompilerParams(
            dimension_semantics=("parallel", "arbitrary")),
    )(cnt, xe, gate_up_proj, gate_up_proj, down_proj)


def _combine_body(oute_hbm, slot0_hbm, slot1_hbm, a0_hbm, a1_hbm, y_hbm,
                  s0_v, s1_v, a0_v, a1_v, r0_v, r1_v, y_v, sem0, sem1):
    wid = lax.axis_index("s") * 2 + lax.axis_index("c")
    tok0 = wid * TOK_W
    pltpu.sync_copy(slot0_hbm.at[pl.ds(tok0, TOK_W)], s0_v)
    pltpu.sync_copy(slot1_hbm.at[pl.ds(tok0, TOK_W)], s1_v)
    pltpu.sync_copy(a0_hbm.at[pl.ds(tok0, TOK_W)], a0_v)
    pltpu.sync_copy(a1_hbm.at[pl.ds(tok0, TOK_W)], a1_v)
    for cidx in range(TOK_W // 16):
        i0 = jnp.minimum(s0_v[pl.ds(cidx * 16, 16)], E * C - 1)
        i1 = jnp.minimum(s1_v[pl.ds(cidx * 16, 16)], E * C - 1)
        cp0 = pltpu.async_copy(oute_hbm.at[i0], r0_v, sem0)
        cp1 = pltpu.async_copy(oute_hbm.at[i1], r1_v, sem1)
        cp0.wait()
        cp1.wait()
        for i in range(16):
            p = cidx * 16 + i
            a0 = a0_v[p, :]
            a1 = a1_v[p, :]
            zero = jnp.zeros((16,), jnp.float32)

            def body(cc, _):
                r0 = r0_v[i, pl.ds(cc * 16, 16)]
                r1 = r1_v[i, pl.ds(cc * 16, 16)]
                y_v[i, pl.ds(cc * 16, 16)] = (
                    jnp.where(a0 > 0, a0 * r0, zero)
                    + jnp.where(a1 > 0, a1 * r1, zero))
                return 0

            lax.fori_loop(0, D // 16, body, 0)
        pltpu.sync_copy(y_v, y_hbm.at[pl.ds(tok0 + cidx * 16, 16)])


def _combine(out_e, slot0, slot1, a0, a1):
    mesh = plsc.VectorSubcoreMesh(core_axis_name="c", subcore_axis_name="s",
                                  num_cores=2, num_subcores=16)
    f = functools.partial(
        pl.kernel,
        out_type=jax.ShapeDtypeStruct((T, D), jnp.float32),
        mesh=mesh,
        scratch_types=[
            pltpu.VMEM((TOK_W,), jnp.int32),
            pltpu.VMEM((TOK_W,), jnp.int32),
            pltpu.VMEM((TOK_W, 16), jnp.float32),
            pltpu.VMEM((TOK_W, 16), jnp.float32),
            pltpu.VMEM((16, D), jnp.float32),
            pltpu.VMEM((16, D), jnp.float32),
            pltpu.VMEM((16, D), jnp.float32),
            pltpu.SemaphoreType.DMA,
            pltpu.SemaphoreType.DMA,
        ],
    )(_combine_body)
    return f(out_e, slot0, slot1, a0, a1)


def kernel(hidden_states, expert_affinities, expert_indices, seq_len,
           gate_up_proj, down_proj):
    del seq_len
    ps0, ps1, a0, a1, cnt = _routing(expert_indices, expert_affinities)
    slot0 = ps0[:, 0]
    slot1 = ps1[:, 0]
    xe = _dispatch(hidden_states, slot0, slot1)
    out_e = _mlp(cnt.reshape(E), xe, gate_up_proj, down_proj)
    return _combine(out_e, slot0, slot1, a0, a1)


def _routing_xla(expert_indices, expert_affinities):
    flat_idx = expert_indices.reshape(-1)
    onehot = jax.nn.one_hot(flat_idx, E, dtype=jnp.int32)
    pos = jnp.sum((jnp.cumsum(onehot, axis=0) - 1) * onehot, axis=1)
    valid = pos < C
    slot = jnp.where(valid, flat_idx * C + pos, SENT).astype(jnp.int32)
    topk = jnp.take_along_axis(expert_affinities, expert_indices, axis=1)
    topk = topk / (jnp.sum(topk, axis=1, keepdims=True) + 1e-9)
    aff = (topk * valid.reshape(T, K)).reshape(-1)
    s2 = slot.reshape(T, K)
    a2 = aff.reshape(T, K)
    a0 = jnp.broadcast_to(a2[:, 0:1], (T, 16))
    a1 = jnp.broadcast_to(a2[:, 1:2], (T, 16))
    return s2[:, 0], s2[:, 1], a0, a1


def _dispatch_xla(hidden, s0, s1):
    xe = jnp.zeros((XROWS, D), jnp.float32)
    xe = xe.at[s0].set(hidden)
    xe = xe.at[s1].set(hidden)
    return xe


def _combine_xla(out_e, s0, s1, a0b, a1b):
    a0 = a0b[:, 0]
    a1 = a1b[:, 0]
    i0 = jnp.minimum(s0, E * C - 1)
    i1 = jnp.minimum(s1, E * C - 1)
    r0 = out_e[i0] * jnp.where(a0 > 0, a0, 0.0)[:, None]
    r1 = out_e[i1] * jnp.where(a1 > 0, a1, 0.0)[:, None]
    return r0 + r1
